# Initial kernel scaffold; baseline (speedup 1.0000x reference)
import jax
import jax.numpy as jnp
from jax.experimental import pallas as pl


def kernel(query, key, value, reference_points, spatial_shapes, level_start_index, W_off, b_off, W_attn, b_attn, W_v, b_v, W_out, b_out):
    raise NotImplementedError("write your pallas kernel here")


# speedup vs baseline: 45.3397x; 45.3397x over previous
import functools

import jax
import jax.numpy as jnp
from jax import lax
from jax.experimental import pallas as pl
from jax.experimental.pallas import tpu as pltpu
from jax.experimental.pallas import tpu_sc as plsc

_DIMS = 256
_HEADS = 8
_POINTS = 4
_QUEUE = 2
_H, _W, _D = 40, 40, 8
_NQ = _H * _W * _D
_HD = _DIMS // _HEADS
_S = _QUEUE * _HEADS * _POINTS
_NC = 8
_SR = _S * _NC

_BQA = 256
_BQC = 64


def _stage_a_body(q_ref, rp_ref, wv_ref, bv_ref, wx_ref, bx_ref, wy_ref,
                  by_ref, wz_ref, bz_ref, wa_ref, ba_ref, vp_ref, idx_ref,
                  wgt_ref):
    q = q_ref[...]
    vp_ref[...] = (jnp.dot(q, wv_ref[...], preferred_element_type=jnp.float32, precision=lax.Precision.HIGHEST)
                   + bv_ref[...])

    logits = jnp.dot(q, wa_ref[...], preferred_element_type=jnp.float32, precision=lax.Precision.HIGHEST) + ba_ref[...]
    logits = logits - jnp.max(logits, axis=-1, keepdims=True)
    e = jnp.exp(logits)
    gi = lax.broadcasted_iota(jnp.int32, (_S, _S), 0) // _POINTS
    gj = lax.broadcasted_iota(jnp.int32, (_S, _S), 1) // _POINTS
    grp = (gi == gj).astype(jnp.float32)
    aw = e / jnp.dot(e, grp, preferred_element_type=jnp.float32, precision=lax.Precision.HIGHEST)

    rp = rp_ref[...]
    qi = lax.broadcasted_iota(jnp.int32, (_QUEUE, _S), 0)
    qj = lax.broadcasted_iota(jnp.int32, (_QUEUE, _S), 1) // (_S // _QUEUE)
    oh = (qi == qj).astype(jnp.float32)
    rx = jnp.dot(rp[:, :, 0], oh, preferred_element_type=jnp.float32, precision=lax.Precision.HIGHEST)
    ry = jnp.dot(rp[:, :, 1], oh, preferred_element_type=jnp.float32, precision=lax.Precision.HIGHEST)
    rz = jnp.dot(rp[:, :, 2], oh, preferred_element_type=jnp.float32, precision=lax.Precision.HIGHEST)
    x = rx * float(_W) + jnp.dot(q, wx_ref[...],
                                 preferred_element_type=jnp.float32, precision=lax.Precision.HIGHEST) + bx_ref[...] - 0.5
    y = ry * float(_H) + jnp.dot(q, wy_ref[...],
                                 preferred_element_type=jnp.float32, precision=lax.Precision.HIGHEST) + by_ref[...] - 0.5
    z = rz * float(_D) + jnp.dot(q, wz_ref[...],
                                 preferred_element_type=jnp.float32, precision=lax.Precision.HIGHEST) + bz_ref[...] - 0.5
    x0f, y0f, z0f = jnp.floor(x), jnp.floor(y), jnp.floor(z)
    x0 = x0f.astype(jnp.int32)
    y0 = y0f.astype(jnp.int32)
    z0 = z0f.astype(jnp.int32)
    fx, fy, fz = x - x0f, y - y0f, z - z0f

    h_lane = (lax.broadcasted_iota(jnp.int32, (_BQA, _S), 1)
              % (_S // _QUEUE)) // _POINTS
    idx_parts = []
    wgt_parts = []
    for dx in (0, 1):
        for dy in (0, 1):
            for dz in (0, 1):
                xi, yi, zi = x0 + dx, y0 + dy, z0 + dz
                w = ((fx if dx else 1.0 - fx) * (fy if dy else 1.0 - fy)
                     * (fz if dz else 1.0 - fz))
                valid = ((xi >= 0) & (xi < _W) & (yi >= 0) & (yi < _H)
                         & (zi >= 0) & (zi < _D))
                w = w * valid.astype(jnp.float32) * aw
                vox = (jnp.clip(yi, 0, _H - 1) * (_W * _D)
                       + jnp.clip(xi, 0, _W - 1) * _D
                       + jnp.clip(zi, 0, _D - 1))
                idx_parts.append(vox * _HEADS + h_lane)
                wgt_parts.append(w)
    idx_ref[...] = jnp.concatenate(idx_parts, axis=-1)
    wgt_ref[...] = jnp.concatenate(wgt_parts, axis=-1)


def _stage_c_body(g_ref, w_ref, q_ref, wout_ref, bout_ref, o_ref):
    g = g_ref[...]
    w = w_ref[...]
    acc = jnp.zeros((_BQC, 128, _HD), dtype=jnp.float32)
    for p in range(_POINTS):
        acc = acc + g[:, :, p * _HD:(p + 1) * _HD] * w[:, :, p:p + 1]
    acc = jnp.sum(acc.reshape(_BQC, _NC, _QUEUE * _HEADS, _HD), axis=1)
    m = 0.5 * (acc[:, :_HEADS] + acc[:, _HEADS:])
    m = m.reshape(_BQC, _DIMS)
    o_ref[...] = (jnp.dot(m, wout_ref[...], preferred_element_type=jnp.float32, precision=lax.Precision.HIGHEST)
                  + bout_ref[...] + q_ref[...])


def _run_stage_a(q2, rp_t, wv_t, b_v, wx, bx, wy, by, wz, bz, wa_t, ba_p,
                 interpret=False):
    nblk = _NQ // _BQA
    full = lambda s: pl.BlockSpec(s, lambda i: (0,) * len(s))
    return pl.pallas_call(
        _stage_a_body,
        grid=(nblk,),
        in_specs=[
            pl.BlockSpec((_BQA, _DIMS), lambda i: (i, 0)),
            pl.BlockSpec((_BQA, _QUEUE, 3), lambda i: (i, 0, 0)),
            full((_DIMS, _DIMS)), full((_DIMS,)),
            full((_DIMS, _S)), full((_S,)),
            full((_DIMS, _S)), full((_S,)),
            full((_DIMS, _S)), full((_S,)),
            full((_DIMS, _S)), full((_S,)),
        ],
        out_specs=[
            pl.BlockSpec((_BQA, _DIMS), lambda i: (i, 0)),
            pl.BlockSpec((_BQA, _SR), lambda i: (i, 0)),
            pl.BlockSpec((_BQA, _SR), lambda i: (i, 0)),
        ],
        out_shape=[
            jax.ShapeDtypeStruct((_NQ, _DIMS), jnp.float32),
            jax.ShapeDtypeStruct((_NQ, _SR), jnp.int32),
            jax.ShapeDtypeStruct((_NQ, _SR), jnp.float32),
        ],
        interpret=interpret,
    )(q2, rp_t, wv_t, b_v, wx, bx, wy, by, wz, bz, wa_t, ba_p)


_NROWS = _NQ * _SR
_CHUNK = 2048


def _sc_gather(table, idx):
    info = plsc.get_sparse_core_info()
    nw = info.num_cores * info.num_subcores
    b_per_w = _NROWS // nw
    n_iter = b_per_w // _CHUNK
    mesh = plsc.VectorSubcoreMesh(core_axis_name="c", subcore_axis_name="s")

    @functools.partial(
        pl.kernel,
        mesh=mesh,
        out_type=jax.ShapeDtypeStruct((_NROWS, _HD), jnp.float32),
        scratch_types=[
            pltpu.VMEM((_CHUNK // 128, 128), jnp.int32),
            pltpu.VMEM((_CHUNK, _HD), jnp.float32),
            pltpu.SemaphoreType.DMA,
        ],
        compiler_params=pltpu.CompilerParams(use_tc_tiling_on_sc=False),
    )
    def k(table_hbm, idx_hbm, out_hbm, idx_v, rows_v, sem):
        wid = lax.axis_index("s") * info.num_cores + lax.axis_index("c")
        k_sub = _CHUNK // 128

        def body(i, _):
            base = wid * b_per_w + i * _CHUNK
            pltpu.sync_copy(idx_hbm.at[pl.ds(base // 128, k_sub)], idx_v)
            handles = [
                pltpu.async_copy(table_hbm.at[idx_v.at[j]],
                                 rows_v.at[pl.ds(j * 128, 128)], sem)
                for j in range(k_sub)
            ]
            for h in handles:
                h.wait()
            pltpu.sync_copy(rows_v, out_hbm.at[pl.ds(base, _CHUNK)])
            return 0

        lax.fori_loop(0, n_iter, body, 0)

    return k(table, idx)


def _run_stage_c(gath, wgt, q2, wout_t, b_out, interpret=False):
    nblk = _NQ // _BQC
    full = lambda s: pl.BlockSpec(s, lambda i: (0,) * len(s))
    return pl.pallas_call(
        _stage_c_body,
        grid=(nblk,),
        in_specs=[
            pl.BlockSpec((_BQC, 128, 128), lambda i: (i, 0, 0)),
            pl.BlockSpec((_BQC, 128, _POINTS), lambda i: (i, 0, 0)),
            pl.BlockSpec((_BQC, _DIMS), lambda i: (i, 0)),
            full((_DIMS, _DIMS)),
            full((_DIMS,)),
        ],
        out_specs=pl.BlockSpec((_BQC, _DIMS), lambda i: (i, 0)),
        out_shape=jax.ShapeDtypeStruct((_NQ, _DIMS), jnp.float32),
        interpret=interpret,
    )(gath, wgt, q2, wout_t, b_out)


def _prep_weights(W_off, b_off, W_attn, b_attn):
    wo_c = W_off[:, :_DIMS] + W_off[:, _DIMS:]
    wa_c = W_attn[:, :_DIMS] + W_attn[:, _DIMS:]
    po = jnp.arange(_S * 3).reshape(_HEADS, _QUEUE, _POINTS, 3).transpose(
        1, 0, 2, 3).reshape(-1)
    pa = jnp.arange(_S).reshape(_HEADS, _QUEUE, _POINTS).transpose(
        1, 0, 2).reshape(-1)
    wo_p = wo_c[po].reshape(_S, 3, _DIMS)
    bo_p = b_off[po].reshape(_S, 3)
    wx, wy, wz = wo_p[:, 0].T, wo_p[:, 1].T, wo_p[:, 2].T
    bx, by, bz = bo_p[:, 0], bo_p[:, 1], bo_p[:, 2]
    return wx, bx, wy, by, wz, bz, wa_c[pa].T, b_attn[pa]


def kernel(query, key, value, reference_points, spatial_shapes,
           level_start_index, W_off, b_off, W_attn, b_attn, W_v, b_v,
           W_out, b_out):
    del key, value, spatial_shapes, level_start_index
    q2 = query[0]
    wx, bx, wy, by, wz, bz, wa_t, ba_p = _prep_weights(W_off, b_off, W_attn,
                                                       b_attn)
    rp_t = reference_points[:, :, 0, :].transpose(1, 0, 2)

    vp, idx, wgt = _run_stage_a(q2, rp_t, W_v.T, b_v, wx, bx, wy, by, wz, bz,
                                wa_t, ba_p)
    table = vp.reshape(_NQ * _HEADS, _HD)
    gath = _sc_gather(table, idx.reshape(_NROWS // 128, 128))
    gath = gath.reshape(_NQ, 128, 128)
    wgt_r = wgt.reshape(_NQ, 128, _POINTS)
    out = _run_stage_c(gath, wgt_r, q2, W_out.T, b_out)
    return out[None]
